# baseline (device time: 101954 ns/iter reference)
import jax
import jax.numpy as jnp
from jax import lax
from jax.experimental import pallas as pl
from jax.experimental.pallas import tpu as pltpu

N_DEV = 4
N_CHUNKS = 4


def kernel(x, w_mat, scale_x, scale_w):
    m_tot, k_per = x.shape
    k_tot, n = w_mat.shape
    m_per = m_tot // N_DEV
    n_chunk = n // N_CHUNKS
    k_rem = (N_DEV - 1) * k_per

    def body(x_ref, w_hbm, sx_ref, sw_ref, out_hbm,
             recv_buf, w_vmem, acc, w_sems, out_sems, send_sems, recv_sems):
        my = lax.axis_index("i")


        rdmas = []
        for off in range(1, N_DEV):
            dst = (my + off) % N_DEV
            rdma = pltpu.make_async_remote_copy(
                src_ref=x_ref.at[pl.ds(dst * m_per, m_per), :],
                dst_ref=recv_buf.at[:, pl.ds((off - 1) * k_per, k_per)],
                send_sem=send_sems.at[off - 1],
                recv_sem=recv_sems.at[off - 1],
                device_id=(dst,),
                device_id_type=pl.DeviceIdType.MESH,
            )
            rdma.start()
            rdmas.append(rdma)

        srcs = [(my - off) % N_DEV for off in range(1, N_DEV)]

        def w1_copy(j, slot):
            return pltpu.make_async_copy(
                w_hbm.at[pl.ds(my * k_per, k_per),
                         pl.ds(j * n_chunk, n_chunk)],
                w_vmem.at[slot, 0:k_per, :],
                w_sems.at[slot, 0],
            )

        def w2_copies(j, slot):
            return [
                pltpu.make_async_copy(
                    w_hbm.at[pl.ds(srcs[i] * k_per, k_per),
                             pl.ds(j * n_chunk, n_chunk)],
                    w_vmem.at[slot, pl.ds(i * k_per, k_per), :],
                    w_sems.at[slot, i],
                )
                for i in range(N_DEV - 1)
            ]

        def out_copy(j):
            ns = pl.ds(j * n_chunk, n_chunk)
            return pltpu.make_async_copy(
                acc.at[:, ns], out_hbm.at[:, ns], out_sems.at[j],
            )

        scale = sx_ref[0] * sw_ref[0]
        x_loc = x_ref.at[pl.ds(my * m_per, m_per), :]

        w1_copy(0, 0).start()
        for j in range(N_CHUNKS):
            slot = j % 2
            nxt = (j + 1) % 2
            if j + 1 < N_CHUNKS:
                w1_copy(j + 1, nxt).start()
            else:
                for c in w2_copies(0, nxt):
                    c.start()
            w1_copy(j, slot).wait()
            acc[:, pl.ds(j * n_chunk, n_chunk)] = jnp.dot(
                x_loc[:, :], w_vmem[slot, 0:k_per, :],
                preferred_element_type=jnp.float32,
            ).astype(jnp.bfloat16)

        for r in rdmas:
            r.wait_recv()
        for j in range(N_CHUNKS):
            slot = (N_CHUNKS + j) % 2
            if j + 1 < N_CHUNKS:
                for c in w2_copies(j + 1, (N_CHUNKS + j + 1) % 2):
                    c.start()
            for c in w2_copies(j, slot):
                c.wait()
            part = jnp.dot(
                recv_buf[:, :], w_vmem[slot],
                preferred_element_type=jnp.float32,
            )
            ns = pl.ds(j * n_chunk, n_chunk)
            acc[:, ns] = (
                (acc[:, ns].astype(jnp.float32) + part) * scale
            ).astype(jnp.bfloat16)
            out_copy(j).start()

        for j in range(N_CHUNKS):
            out_copy(j).wait()

        for r in rdmas:
            r.wait_send()

    return pl.pallas_call(
        body,
        out_shape=jax.ShapeDtypeStruct((m_per, n), jnp.bfloat16),
        in_specs=[
            pl.BlockSpec(memory_space=pltpu.MemorySpace.VMEM),
            pl.BlockSpec(memory_space=pltpu.MemorySpace.HBM),
            pl.BlockSpec(memory_space=pltpu.MemorySpace.SMEM),
            pl.BlockSpec(memory_space=pltpu.MemorySpace.SMEM),
        ],
        out_specs=pl.BlockSpec(memory_space=pltpu.MemorySpace.HBM),
        scratch_shapes=[
            pltpu.VMEM((m_per, k_rem), jnp.int8),
            pltpu.VMEM((2, k_rem, n_chunk), jnp.int8),
            pltpu.VMEM((m_per, n), jnp.bfloat16),
            pltpu.SemaphoreType.DMA((2, N_DEV - 1)),
            pltpu.SemaphoreType.DMA((N_CHUNKS,)),
            pltpu.SemaphoreType.DMA((N_DEV - 1,)),
            pltpu.SemaphoreType.DMA((N_DEV - 1,)),
        ],
        compiler_params=pltpu.CompilerParams(
            vmem_limit_bytes=60 * 1024 * 1024,
        ),
    )(x, w_mat, scale_x, scale_w)


# device time: 101819 ns/iter; 1.0013x vs baseline; 1.0013x over previous
import jax
import jax.numpy as jnp
from jax import lax
from jax.experimental import pallas as pl
from jax.experimental.pallas import tpu as pltpu

N_DEV = 4
N_CHUNKS = 4


def kernel(x, w_mat, scale_x, scale_w):
    m_tot, k_per = x.shape
    k_tot, n = w_mat.shape
    m_per = m_tot // N_DEV
    n_chunk = n // N_CHUNKS
    k_rem = (N_DEV - 1) * k_per

    def body(x_ref, w_hbm, sx_ref, sw_ref, out_hbm,
             recv_buf, w_vmem, acc, w_sems, out_sems, send_sems, recv_sems):
        my = lax.axis_index("i")


        rdmas = []
        for off in range(1, N_DEV):
            dst = (my + off) % N_DEV
            rdma = pltpu.make_async_remote_copy(
                src_ref=x_ref.at[pl.ds(dst * m_per, m_per), :],
                dst_ref=recv_buf.at[:, pl.ds((off - 1) * k_per, k_per)],
                send_sem=send_sems.at[off - 1],
                recv_sem=recv_sems.at[off - 1],
                device_id=(dst,),
                device_id_type=pl.DeviceIdType.MESH,
            )
            rdma.start()
            rdmas.append(rdma)

        srcs = [(my - off) % N_DEV for off in range(1, N_DEV)]

        def w1_copy(j, slot):
            return pltpu.make_async_copy(
                w_hbm.at[pl.ds(my * k_per, k_per),
                         pl.ds(j * n_chunk, n_chunk)],
                w_vmem.at[slot, 0:k_per, :],
                w_sems.at[slot, 0],
            )

        def w2_copies(j, slot):
            return [
                pltpu.make_async_copy(
                    w_hbm.at[pl.ds(srcs[i] * k_per, k_per),
                             pl.ds(j * n_chunk, n_chunk)],
                    w_vmem.at[slot, pl.ds(i * k_per, k_per), :],
                    w_sems.at[slot, i],
                )
                for i in range(N_DEV - 1)
            ]

        def out_copy(j):
            ns = pl.ds(j * n_chunk, n_chunk)
            return pltpu.make_async_copy(
                acc.at[:, ns], out_hbm.at[:, ns], out_sems.at[j],
            )

        scale = sx_ref[0] * sw_ref[0]
        x_loc = x_ref.at[pl.ds(my * m_per, m_per), :]

        def start_copies(s):
            if s < N_CHUNKS:
                w1_copy(s, s % 3).start()
            else:
                for c in w2_copies(s - N_CHUNKS, s % 3):
                    c.start()

        start_copies(0)
        start_copies(1)

        for j in range(N_CHUNKS):
            slot = j % 3
            start_copies(j + 2)
            w1_copy(j, slot).wait()
            acc[:, pl.ds(j * n_chunk, n_chunk)] = jnp.dot(
                x_loc[:, :], w_vmem[slot, 0:k_per, :],
                preferred_element_type=jnp.float32,
            ).astype(jnp.bfloat16)

        for r in rdmas:
            r.wait_recv()
        for j in range(N_CHUNKS):
            s = N_CHUNKS + j
            slot = s % 3
            if s + 2 < 2 * N_CHUNKS:
                start_copies(s + 2)
            for c in w2_copies(j, slot):
                c.wait()
            part = jnp.dot(
                recv_buf[:, :], w_vmem[slot],
                preferred_element_type=jnp.float32,
            )
            ns = pl.ds(j * n_chunk, n_chunk)
            acc[:, ns] = (
                (acc[:, ns].astype(jnp.float32) + part) * scale
            ).astype(jnp.bfloat16)
            out_copy(j).start()

        for j in range(N_CHUNKS):
            out_copy(j).wait()

        for r in rdmas:
            r.wait_send()

    return pl.pallas_call(
        body,
        out_shape=jax.ShapeDtypeStruct((m_per, n), jnp.bfloat16),
        in_specs=[
            pl.BlockSpec(memory_space=pltpu.MemorySpace.VMEM),
            pl.BlockSpec(memory_space=pltpu.MemorySpace.HBM),
            pl.BlockSpec(memory_space=pltpu.MemorySpace.SMEM),
            pl.BlockSpec(memory_space=pltpu.MemorySpace.SMEM),
        ],
        out_specs=pl.BlockSpec(memory_space=pltpu.MemorySpace.HBM),
        scratch_shapes=[
            pltpu.VMEM((m_per, k_rem), jnp.int8),
            pltpu.VMEM((3, k_rem, n_chunk), jnp.int8),
            pltpu.VMEM((m_per, n), jnp.bfloat16),
            pltpu.SemaphoreType.DMA((3, N_DEV - 1)),
            pltpu.SemaphoreType.DMA((N_CHUNKS,)),
            pltpu.SemaphoreType.DMA((N_DEV - 1,)),
            pltpu.SemaphoreType.DMA((N_DEV - 1,)),
        ],
        compiler_params=pltpu.CompilerParams(
            vmem_limit_bytes=60 * 1024 * 1024,
        ),
    )(x, w_mat, scale_x, scale_w)


# device time: 99236 ns/iter; 1.0274x vs baseline; 1.0260x over previous
import jax
import jax.numpy as jnp
from jax import lax
from jax.experimental import pallas as pl
from jax.experimental.pallas import tpu as pltpu

N_DEV = 4
N_CHUNKS = 4


def kernel(x, w_mat, scale_x, scale_w):
    m_tot, k_per = x.shape
    k_tot, n = w_mat.shape
    m_per = m_tot // N_DEV
    n_chunk = n // N_CHUNKS

    def body(x_ref, w_hbm, sx_ref, sw_ref, out_hbm,
             recv_buf, w_vmem, acc, w_sems, out_sems, send_sems, recv_sems):
        my = lax.axis_index("i")


        rdmas = []
        for off in range(1, N_DEV):
            dst = (my + off) % N_DEV
            rdma = pltpu.make_async_remote_copy(
                src_ref=x_ref.at[pl.ds(dst * m_per, m_per), :],
                dst_ref=recv_buf.at[off - 1],
                send_sem=send_sems.at[off - 1],
                recv_sem=recv_sems.at[off - 1],
                device_id=(dst,),
                device_id_type=pl.DeviceIdType.MESH,
            )
            rdma.start()
            rdmas.append(rdma)

        offs = (0, 1, 3, 2)
        srcs = [(my - off) % N_DEV for off in offs]
        steps = [(c, j) for c in range(N_DEV) for j in range(N_CHUNKS)]

        def w_copy(step, slot):
            c, j = steps[step]
            return pltpu.make_async_copy(
                w_hbm.at[pl.ds(srcs[c] * k_per, k_per),
                         pl.ds(j * n_chunk, n_chunk)],
                w_vmem.at[slot],
                w_sems.at[slot],
            )

        def out_copy(j):
            ns = pl.ds(j * n_chunk, n_chunk)
            return pltpu.make_async_copy(
                acc.at[:, ns], out_hbm.at[:, ns], out_sems.at[j],
            )

        scale = sx_ref[0] * sw_ref[0]

        w_copy(0, 0).start()
        for s, (c, j) in enumerate(steps):
            slot = s % 2
            if s + 1 < len(steps):
                w_copy(s + 1, (s + 1) % 2).start()
            w_copy(s, slot).wait()
            if offs[c] == 0:
                x_blk = x_ref[pl.ds(my * m_per, m_per), :]
            else:
                if j == 0:
                    rdmas[[1, 3, 2].index(offs[c])].wait_recv()
                x_blk = recv_buf[offs[c] - 1]
            part = jnp.dot(
                x_blk, w_vmem[slot], preferred_element_type=jnp.float32
            )
            ns = pl.ds(j * n_chunk, n_chunk)
            if c == 0:
                acc[:, ns] = part.astype(jnp.bfloat16)
            elif c == N_DEV - 1:
                acc[:, ns] = (
                    (acc[:, ns].astype(jnp.float32) + part) * scale
                ).astype(jnp.bfloat16)
                out_copy(j).start()
            else:
                acc[:, ns] = (
                    acc[:, ns].astype(jnp.float32) + part
                ).astype(jnp.bfloat16)

        for j in range(N_CHUNKS):
            out_copy(j).wait()

        for r in rdmas:
            r.wait_send()

    return pl.pallas_call(
        body,
        out_shape=jax.ShapeDtypeStruct((m_per, n), jnp.bfloat16),
        in_specs=[
            pl.BlockSpec(memory_space=pltpu.MemorySpace.VMEM),
            pl.BlockSpec(memory_space=pltpu.MemorySpace.HBM),
            pl.BlockSpec(memory_space=pltpu.MemorySpace.SMEM),
            pl.BlockSpec(memory_space=pltpu.MemorySpace.SMEM),
        ],
        out_specs=pl.BlockSpec(memory_space=pltpu.MemorySpace.HBM),
        scratch_shapes=[
            pltpu.VMEM((N_DEV - 1, m_per, k_per), jnp.int8),
            pltpu.VMEM((2, k_per, n_chunk), jnp.int8),
            pltpu.VMEM((m_per, n), jnp.bfloat16),
            pltpu.SemaphoreType.DMA((2,)),
            pltpu.SemaphoreType.DMA((N_CHUNKS,)),
            pltpu.SemaphoreType.DMA((N_DEV - 1,)),
            pltpu.SemaphoreType.DMA((N_DEV - 1,)),
        ],
        compiler_params=pltpu.CompilerParams(
            vmem_limit_bytes=60 * 1024 * 1024,
        ),
    )(x, w_mat, scale_x, scale_w)
